# baseline (speedup 1.0000x reference)
import functools

import jax
import jax.numpy as jnp
from jax import lax
from jax.experimental import pallas as pl
from jax.experimental.pallas import tpu as pltpu
from jax.experimental.pallas import tpu_sc as plsc

MAX_GLYPH = 5976
N_GROUPS = 16
GROUP_DIV = 384
HW = 81
P = 84
N_HP_BUCKETS = 32
N_HUNGER = 7
N_COND = 13
D = 512
B = 2048

HP_OFF = MAX_GLYPH
HUNGER_OFF = HP_OFF + 32
COND_OFF = HUNGER_OFF + 8
TAB_ROWS = COND_OFF + B

NW = 32
CH = 32
NPIECE = 6
RPIECE = (B * P) // NPIECE
RPW = RPIECE // NW
NCH = RPW // CH
NBUF = 4


DH = D // 2


def _rn16(x):
    u = lax.bitcast_convert_type(x, jnp.uint32)
    return (u + jnp.uint32(0x7FFF) + ((u >> 16) & jnp.uint32(1))) >> 16


def _pack_pair(x):
    lo = _rn16(x[..., :DH])
    hi = _rn16(x[..., DH:])
    return lax.bitcast_convert_type(lo | (hi << 16), jnp.int32)


def _fuse_body(ent_ref, grp_ref, out_ref):
    k = pl.program_id(0)
    out_ref[...] = _pack_pair(ent_ref[...] + grp_ref[pl.ds(k, 1), :])


def _fuse_table(entity_emb, group_emb):
    return pl.pallas_call(
        _fuse_body,
        grid=(N_GROUPS,),
        in_specs=[
            pl.BlockSpec((GROUP_DIV, D), lambda k: (k, 0)),
            pl.BlockSpec((N_GROUPS, D), lambda k: (0, 0)),
        ],
        out_specs=pl.BlockSpec((GROUP_DIV, DH), lambda k: (k, 0)),
        out_shape=jax.ShapeDtypeStruct((MAX_GLYPH, DH), jnp.int32),
    )(entity_emb, group_emb)


_PREP_BLK = 256


def _prep_body(vic_ref, bl_ref, cond_ref, idx_ref, econd_ref):
    pid = pl.program_id(0)
    hp = bl_ref[:, 10:11]
    maxhp = bl_ref[:, 11:12]
    num = (hp * N_HP_BUCKETS).astype(jnp.float32)
    den = jnp.maximum(maxhp, 1).astype(jnp.float32)
    bucket = jnp.clip(jnp.floor(num / den).astype(jnp.int32), 0, N_HP_BUCKETS - 1)
    hung = lax.rem(bl_ref[:, 21:22], N_HUNGER)
    rowid = pid * _PREP_BLK + lax.broadcasted_iota(jnp.int32, (_PREP_BLK, 1), 0)
    idx_ref[...] = jnp.concatenate(
        [vic_ref[...], HP_OFF + bucket, HUNGER_OFF + hung, COND_OFF + rowid],
        axis=1,
    )
    cond = bl_ref[:, 25:26]
    shifts = lax.broadcasted_iota(jnp.int32, (_PREP_BLK, N_COND), 1)
    bits = ((cond >> shifts) & 1).astype(jnp.float32)
    acc = jnp.zeros((_PREP_BLK, D), jnp.float32)
    for k in range(N_COND):
        acc = acc + bits[:, k : k + 1] * cond_ref[k : k + 1, :]
    econd_ref[...] = _pack_pair(acc)


def _prep(vicinity2d, blstats, cond_emb):
    grid = (B // _PREP_BLK,)
    return pl.pallas_call(
        _prep_body,
        grid=grid,
        in_specs=[
            pl.BlockSpec((_PREP_BLK, HW), lambda k: (k, 0)),
            pl.BlockSpec((_PREP_BLK, 27), lambda k: (k, 0)),
            pl.BlockSpec((N_COND, D), lambda k: (0, 0)),
        ],
        out_specs=[
            pl.BlockSpec((_PREP_BLK, P), lambda k: (k, 0)),
            pl.BlockSpec((_PREP_BLK, DH), lambda k: (k, 0)),
        ],
        out_shape=[
            jax.ShapeDtypeStruct((B, P), jnp.int32),
            jax.ShapeDtypeStruct((B, DH), jnp.int32),
        ],
    )(vicinity2d, blstats, cond_emb)


def _sc_body(
    idx_hbm, table_hbm, out_hbm,
    idx_all, rows, gsems, osems,
):
    c = lax.axis_index("c")
    s = lax.axis_index("s")
    wid = s * 2 + c
    base = wid * RPW
    pltpu.sync_copy(idx_hbm.at[pl.ds(base, RPW)], idx_all)

    def gather(i, b):
        return pltpu.make_async_copy(
            table_hbm.at[idx_all.at[pl.ds(i * CH, CH)]], rows[b], gsems[b]
        )

    def out_copy(i, b):
        return pltpu.make_async_copy(
            rows[b], out_hbm.at[pl.ds(base + i * CH, CH), :], osems[b]
        )

    for b in range(NBUF):
        gather(b, b).start()

    def quad(i4, carry):
        for par in range(NBUF):
            i = i4 * NBUF + par
            gather(i, par).wait()
            out_copy(i, par).start()

            @pl.when(i >= 1)
            def _():
                out_copy(i - 1, (par - 1) % NBUF).wait()

            @pl.when((i >= 1) & (i + NBUF - 1 < NCH))
            def _():
                gather(i + NBUF - 1, (par - 1) % NBUF).start()
        return carry

    lax.fori_loop(0, NCH // NBUF, quad, 0)
    out_copy(NCH - 1, (NCH - 1) % NBUF).wait()


_FBB = 2048


_PBLK = RPIECE // _FBB


def _finish_piece(piece):
    off = piece * _PBLK

    def compute(g_ref, ego_ref, out_ref):
        p = (pl.program_id(0) + off) // (B // _FBB)
        u = lax.bitcast_convert_type(g_ref[...], jnp.uint32)
        a = lax.bitcast_convert_type(u << 16, jnp.float32)
        bh = lax.bitcast_convert_type(u & jnp.uint32(0xFFFF0000), jnp.float32)
        e = ego_ref[pl.ds(p, 1), :]
        out_ref[:, :DH] = a + e[:, :DH]
        out_ref[:, DH:] = bh + e[:, DH:]

    g_spec = pl.BlockSpec((_FBB, DH), lambda k: (k, 0))
    ego_spec = pl.BlockSpec((P, D), lambda k: (0, 0))
    out_spec = pl.BlockSpec((_FBB, D), lambda k: (k + off, 0))
    out_sh = jax.ShapeDtypeStruct((B * P, D), jnp.float32)
    if piece == 0:
        return pl.pallas_call(
            compute,
            grid=(_PBLK,),
            in_specs=[g_spec, ego_spec],
            out_specs=out_spec,
            out_shape=out_sh,
        )

    def body(buf_ref, g_ref, ego_ref, out_ref):
        compute(g_ref, ego_ref, out_ref)

    return pl.pallas_call(
        body,
        grid=(_PBLK,),
        in_specs=[pl.BlockSpec(memory_space=pl.ANY), g_spec, ego_spec],
        out_specs=out_spec,
        out_shape=out_sh,
        input_output_aliases={0: 0},
    )


@functools.cache
def _sc_gather():
    return pl.kernel(
        _sc_body,
        out_type=jax.ShapeDtypeStruct((RPIECE, DH), jnp.int32),
        mesh=plsc.VectorSubcoreMesh(core_axis_name="c", subcore_axis_name="s"),
        scratch_types=[
            pltpu.VMEM((RPW,), jnp.int32),
            [pltpu.VMEM((CH, DH), jnp.int32)] * NBUF,
            [pltpu.SemaphoreType.DMA] * NBUF,
            [pltpu.SemaphoreType.DMA] * NBUF,
        ],
    )


def kernel(vicinity, blstats, entity_emb, group_emb, ego_emb, hp_emb, hunger_emb, cond_emb):
    vic2d = vicinity.reshape(B, HW).astype(jnp.int32)
    fused = _fuse_table(entity_emb, group_emb)
    idx, e_cond = _prep(vic2d, blstats, cond_emb)
    small = jnp.concatenate(
        [hp_emb, hunger_emb, jnp.zeros((1, D), jnp.float32)], axis=0
    )
    lo = lax.bitcast_convert_type(small[:, :DH].astype(jnp.bfloat16), jnp.uint16)
    hi = lax.bitcast_convert_type(small[:, DH:].astype(jnp.bfloat16), jnp.uint16)
    small_p = lax.bitcast_convert_type(
        lo.astype(jnp.uint32) | (hi.astype(jnp.uint32) << 16), jnp.int32
    )
    table = jnp.concatenate([fused, small_p, e_cond], axis=0)
    ego_ext = jnp.concatenate(
        [ego_emb.reshape(HW, D), jnp.zeros((P - HW, D), jnp.float32)], axis=0
    )
    idx_pm = idx.T.reshape(B * P)
    gs = [
        _sc_gather()(lax.slice_in_dim(idx_pm, k * RPIECE, (k + 1) * RPIECE), table)
        for k in range(NPIECE)
    ]
    buf = _finish_piece(0)(gs[0], ego_ext)
    for k in range(1, NPIECE):
        buf = _finish_piece(k)(buf, gs[k], ego_ext)
    return buf.reshape(P, B, D).transpose(1, 0, 2)


# speedup vs baseline: 1.0060x; 1.0060x over previous
import functools

import jax
import jax.numpy as jnp
from jax import lax
from jax.experimental import pallas as pl
from jax.experimental.pallas import tpu as pltpu
from jax.experimental.pallas import tpu_sc as plsc

MAX_GLYPH = 5976
N_GROUPS = 16
GROUP_DIV = 384
HW = 81
P = 84
N_HP_BUCKETS = 32
N_HUNGER = 7
N_COND = 13
D = 512
B = 2048

HP_OFF = MAX_GLYPH
HUNGER_OFF = HP_OFF + 32
COND_OFF = HUNGER_OFF + 8
TAB_ROWS = COND_OFF + B

NW = 32
CH = 48
NPIECE = 4
RPIECE = (B * P) // NPIECE
RPW = RPIECE // NW
NCH = RPW // CH
NBUF = 4


DH = D // 2


def _rn16(x):
    u = lax.bitcast_convert_type(x, jnp.uint32)
    return (u + jnp.uint32(0x7FFF) + ((u >> 16) & jnp.uint32(1))) >> 16


def _pack_pair(x):
    lo = _rn16(x[..., :DH])
    hi = _rn16(x[..., DH:])
    return lax.bitcast_convert_type(lo | (hi << 16), jnp.int32)


def _fuse_body(ent_ref, grp_ref, out_ref):
    k = pl.program_id(0)
    out_ref[...] = _pack_pair(ent_ref[...] + grp_ref[pl.ds(k, 1), :])


def _fuse_table(entity_emb, group_emb):
    return pl.pallas_call(
        _fuse_body,
        grid=(N_GROUPS,),
        in_specs=[
            pl.BlockSpec((GROUP_DIV, D), lambda k: (k, 0)),
            pl.BlockSpec((N_GROUPS, D), lambda k: (0, 0)),
        ],
        out_specs=pl.BlockSpec((GROUP_DIV, DH), lambda k: (k, 0)),
        out_shape=jax.ShapeDtypeStruct((MAX_GLYPH, DH), jnp.int32),
    )(entity_emb, group_emb)


_PREP_BLK = 256


def _prep_body(vic_ref, bl_ref, cond_ref, idx_ref, econd_ref):
    pid = pl.program_id(0)
    hp = bl_ref[:, 10:11]
    maxhp = bl_ref[:, 11:12]
    num = (hp * N_HP_BUCKETS).astype(jnp.float32)
    den = jnp.maximum(maxhp, 1).astype(jnp.float32)
    bucket = jnp.clip(jnp.floor(num / den).astype(jnp.int32), 0, N_HP_BUCKETS - 1)
    hung = lax.rem(bl_ref[:, 21:22], N_HUNGER)
    rowid = pid * _PREP_BLK + lax.broadcasted_iota(jnp.int32, (_PREP_BLK, 1), 0)
    idx_ref[...] = jnp.concatenate(
        [vic_ref[...], HP_OFF + bucket, HUNGER_OFF + hung, COND_OFF + rowid],
        axis=1,
    )
    cond = bl_ref[:, 25:26]
    shifts = lax.broadcasted_iota(jnp.int32, (_PREP_BLK, N_COND), 1)
    bits = ((cond >> shifts) & 1).astype(jnp.float32)
    acc = jnp.zeros((_PREP_BLK, D), jnp.float32)
    for k in range(N_COND):
        acc = acc + bits[:, k : k + 1] * cond_ref[k : k + 1, :]
    econd_ref[...] = _pack_pair(acc)


def _prep(vicinity2d, blstats, cond_emb):
    grid = (B // _PREP_BLK,)
    return pl.pallas_call(
        _prep_body,
        grid=grid,
        in_specs=[
            pl.BlockSpec((_PREP_BLK, HW), lambda k: (k, 0)),
            pl.BlockSpec((_PREP_BLK, 27), lambda k: (k, 0)),
            pl.BlockSpec((N_COND, D), lambda k: (0, 0)),
        ],
        out_specs=[
            pl.BlockSpec((_PREP_BLK, P), lambda k: (k, 0)),
            pl.BlockSpec((_PREP_BLK, DH), lambda k: (k, 0)),
        ],
        out_shape=[
            jax.ShapeDtypeStruct((B, P), jnp.int32),
            jax.ShapeDtypeStruct((B, DH), jnp.int32),
        ],
    )(vicinity2d, blstats, cond_emb)


def _sc_body(
    idx_hbm, table_hbm, out_hbm,
    idx_all, rows, gsems, osems,
):
    c = lax.axis_index("c")
    s = lax.axis_index("s")
    wid = s * 2 + c
    base = wid * RPW
    pltpu.sync_copy(idx_hbm.at[pl.ds(base, RPW)], idx_all)

    def gather(i, b):
        return pltpu.make_async_copy(
            table_hbm.at[idx_all.at[pl.ds(i * CH, CH)]], rows[b], gsems[b]
        )

    def out_copy(i, b):
        return pltpu.make_async_copy(
            rows[b], out_hbm.at[pl.ds(base + i * CH, CH), :], osems[b]
        )

    for b in range(NBUF):
        gather(b, b).start()

    def quad(i4, carry):
        for par in range(NBUF):
            i = i4 * NBUF + par
            gather(i, par).wait()
            out_copy(i, par).start()

            @pl.when(i >= 1)
            def _():
                out_copy(i - 1, (par - 1) % NBUF).wait()

            @pl.when((i >= 1) & (i + NBUF - 1 < NCH))
            def _():
                gather(i + NBUF - 1, (par - 1) % NBUF).start()
        return carry

    lax.fori_loop(0, NCH // NBUF, quad, 0)
    out_copy(NCH - 1, (NCH - 1) % NBUF).wait()


_FBB = 2048


_PBLK = RPIECE // _FBB


def _finish_piece(piece):
    off = piece * _PBLK

    def compute(g_ref, ego_ref, out_ref):
        p = (pl.program_id(0) + off) // (B // _FBB)
        u = lax.bitcast_convert_type(g_ref[...], jnp.uint32)
        a = lax.bitcast_convert_type(u << 16, jnp.float32)
        bh = lax.bitcast_convert_type(u & jnp.uint32(0xFFFF0000), jnp.float32)
        e = ego_ref[pl.ds(p, 1), :]
        out_ref[:, :DH] = a + e[:, :DH]
        out_ref[:, DH:] = bh + e[:, DH:]

    g_spec = pl.BlockSpec((_FBB, DH), lambda k: (k, 0))
    ego_spec = pl.BlockSpec((P, D), lambda k: (0, 0))
    out_spec = pl.BlockSpec((_FBB, D), lambda k: (k + off, 0))
    out_sh = jax.ShapeDtypeStruct((B * P, D), jnp.float32)
    if piece == 0:
        return pl.pallas_call(
            compute,
            grid=(_PBLK,),
            in_specs=[g_spec, ego_spec],
            out_specs=out_spec,
            out_shape=out_sh,
        )

    def body(buf_ref, g_ref, ego_ref, out_ref):
        compute(g_ref, ego_ref, out_ref)

    return pl.pallas_call(
        body,
        grid=(_PBLK,),
        in_specs=[pl.BlockSpec(memory_space=pl.ANY), g_spec, ego_spec],
        out_specs=out_spec,
        out_shape=out_sh,
        input_output_aliases={0: 0},
    )


@functools.cache
def _sc_gather():
    return pl.kernel(
        _sc_body,
        out_type=jax.ShapeDtypeStruct((RPIECE, DH), jnp.int32),
        mesh=plsc.VectorSubcoreMesh(core_axis_name="c", subcore_axis_name="s"),
        scratch_types=[
            pltpu.VMEM((RPW,), jnp.int32),
            [pltpu.VMEM((CH, DH), jnp.int32)] * NBUF,
            [pltpu.SemaphoreType.DMA] * NBUF,
            [pltpu.SemaphoreType.DMA] * NBUF,
        ],
    )


def kernel(vicinity, blstats, entity_emb, group_emb, ego_emb, hp_emb, hunger_emb, cond_emb):
    vic2d = vicinity.reshape(B, HW).astype(jnp.int32)
    fused = _fuse_table(entity_emb, group_emb)
    idx, e_cond = _prep(vic2d, blstats, cond_emb)
    small = jnp.concatenate(
        [hp_emb, hunger_emb, jnp.zeros((1, D), jnp.float32)], axis=0
    )
    lo = lax.bitcast_convert_type(small[:, :DH].astype(jnp.bfloat16), jnp.uint16)
    hi = lax.bitcast_convert_type(small[:, DH:].astype(jnp.bfloat16), jnp.uint16)
    small_p = lax.bitcast_convert_type(
        lo.astype(jnp.uint32) | (hi.astype(jnp.uint32) << 16), jnp.int32
    )
    table = jnp.concatenate([fused, small_p, e_cond], axis=0)
    ego_ext = jnp.concatenate(
        [ego_emb.reshape(HW, D), jnp.zeros((P - HW, D), jnp.float32)], axis=0
    )
    idx_pm = idx.T.reshape(B * P)
    gs = [
        _sc_gather()(lax.slice_in_dim(idx_pm, k * RPIECE, (k + 1) * RPIECE), table)
        for k in range(NPIECE)
    ]
    buf = _finish_piece(0)(gs[0], ego_ext)
    for k in range(1, NPIECE):
        buf = _finish_piece(k)(buf, gs[k], ego_ext)
    return buf.reshape(P, B, D).transpose(1, 0, 2)
